# Initial kernel scaffold; baseline (speedup 1.0000x reference)
import jax
import jax.numpy as jnp
from jax.experimental import pallas as pl


def kernel(x, edge_index, batch, W_agg0, b_agg0, W_lin0, b_lin0, W_agg1, b_agg1, W_lin1, b_lin1, W_post1, b_post1, W_post2, b_post2):
    raise NotImplementedError("write your pallas kernel here")


# speedup vs baseline: 2.6115x; 2.6115x over previous
import functools

import jax
import jax.numpy as jnp
from jax import lax
from jax.experimental import pallas as pl
from jax.experimental.pallas import tpu as pltpu
from jax.experimental.pallas import tpu_sc as plsc

N = 10000
E = 320000
D = 128
NC = 2
NS = 16
NW = NC * NS
CHUNK = 128
EPW = 10240
E_PAD = NW * EPW
NP8 = N + 8
CW = 16

_f32 = jnp.float32
_MESH = plsc.VectorSubcoreMesh(
    core_axis_name="c", subcore_axis_name="s", num_cores=NC, num_subcores=NS
)


def _make_sc_segment_sum(with_counts, ec):
    cpw = EPW // ec
    out_types = [jax.ShapeDtypeStruct((NC, NP8, D), _f32)]
    scratch = [
        pltpu.VMEM((ec,), jnp.int32),
        pltpu.VMEM((ec,), jnp.int32),
        pltpu.VMEM((ec, D), _f32),
        pltpu.VMEM_SHARED((NP8, D), _f32),
        pltpu.SemaphoreType.DMA,
    ]
    if with_counts:
        out_types.append(jax.ShapeDtypeStruct((NC, NP8, D), _f32))
        scratch += [
            pltpu.VMEM((ec, D), _f32),
        ]

    def body(*refs):
        if with_counts:
            (table, srcp, dstp, psum, pcnt,
             sidx, didx, rows, acc, sem, onesw) = refs
        else:
            table, srcp, dstp, psum, sidx, didx, rows, acc, sem = refs
        c = lax.axis_index("c")
        s = lax.axis_index("s")
        w = c * NS + s
        zero16 = jnp.zeros((16,), _f32)
        one16 = jnp.ones((16,), _f32)

        @pl.loop(0, ec)
        def _(i):
            @pl.loop(0, D // 16)
            def _(j):
                rows[i, pl.ds(j * 16, 16)] = zero16

        if with_counts:
            @pl.loop(0, ec)
            def _(i):
                @pl.loop(0, D // 16)
                def _(j):
                    onesw[i, pl.ds(j * 16, 16)] = one16

        def inner():
            zstart = s * 624
            nrows_lo, nrows_hi = 624, 648
            def zero_stripe(zsrc, nrows):
                done = 0
                while done < nrows:
                    step = min(ec, nrows - done)
                    pltpu.sync_copy(zsrc.at[pl.ds(0, step)],
                                    acc.at[pl.ds(zstart + done, step)])
                    done += step
            def zero_mine(zsrc):
                @pl.when(s < NS - 1)
                def _():
                    zero_stripe(zsrc, nrows_lo)
                @pl.when(s == NS - 1)
                def _():
                    zero_stripe(zsrc, nrows_hi)

            def drain_stripe(out_hbm, nrows):
                done = 0
                while done < nrows:
                    step = min(ec, nrows - done)
                    pltpu.sync_copy(acc.at[pl.ds(zstart + done, step)],
                                    rows.at[pl.ds(0, step)])
                    pltpu.sync_copy(rows.at[pl.ds(0, step)],
                                    out_hbm.at[c, pl.ds(zstart + done, step)])
                    done += step

            def drain_mine(out_hbm):
                @pl.when(s < NS - 1)
                def _():
                    drain_stripe(out_hbm, nrows_lo)
                @pl.when(s == NS - 1)
                def _():
                    drain_stripe(out_hbm, nrows_hi)

            zero_mine(rows)
            plsc.subcore_barrier()

            base = w * EPW

            @pl.loop(0, cpw)
            def _(k):
                off = base + k * ec
                pltpu.sync_copy(srcp.at[pl.ds(off, ec)], sidx)
                pltpu.sync_copy(dstp.at[pl.ds(off, ec)], didx)
                pltpu.async_copy(table.at[sidx], rows, sem).wait()
                pltpu.sync_copy(rows, acc.at[didx], add=True)

            plsc.subcore_barrier()
            drain_mine(psum)

            if with_counts:
                @pl.loop(0, ec)
                def _(i):
                    @pl.loop(0, D // 16)
                    def _(j):
                        rows[i, pl.ds(j * 16, 16)] = zero16

                zero_mine(rows)
                plsc.subcore_barrier()

                @pl.loop(0, cpw)
                def _(k):
                    off = base + k * ec
                    pltpu.sync_copy(dstp.at[pl.ds(off, ec)], didx)
                    pltpu.sync_copy(onesw, acc.at[didx], add=True)

                plsc.subcore_barrier()
                drain_mine(pcnt)

        inner()

    return pl.kernel(
        body,
        out_type=tuple(out_types) if with_counts else out_types[0],
        mesh=_MESH,
        scratch_types=scratch,
    )


_sc_seg_cnt = _make_sc_segment_sum(with_counts=True, ec=64)
_sc_seg = _make_sc_segment_sum(with_counts=False, ec=128)

_DOT = dict(precision=lax.Precision.HIGHEST, preferred_element_type=_f32)


def _dot_t(a, w):
    return lax.dot_general(a, w, (((1,), (1,)), ((), ())), **_DOT)


def _tc_pre_body(x_ref, w_ref, b_ref, o_ref):
    o_ref[...] = jnp.maximum(_dot_t(x_ref[...], w_ref[...]) + b_ref[...], 0.0)


def _tc_pre(x, w, b):
    return pl.pallas_call(
        _tc_pre_body, out_shape=jax.ShapeDtypeStruct((N, D), _f32)
    )(x, w, b)


def _combine_update(ps_ref, pc_ref, x_lin, b):
    cnt = (pc_ref[0, pl.ds(0, N), 0:1] + pc_ref[1, pl.ds(0, N), 0:1])
    agg = ((ps_ref[0, pl.ds(0, N)] + ps_ref[1, pl.ds(0, N)])
           / jnp.maximum(cnt, 1.0))
    h = jnp.maximum(agg + x_lin + b, 0.0)
    nrm = jnp.sqrt(jnp.sum(h * h, axis=1, keepdims=True))
    h = h / jnp.maximum(nrm, 1e-12)
    return jnp.maximum(h, 0.0)


def _tc_mid_body(ps_ref, pc_ref, x_ref, wl_ref, bl_ref, wa_ref, ba_ref,
                 hr_ref, o1_ref):
    h = _combine_update(ps_ref, pc_ref, _dot_t(x_ref[...], wl_ref[...]),
                        bl_ref[...])
    hr_ref[...] = h
    o1_ref[...] = jnp.maximum(_dot_t(h, wa_ref[...]) + ba_ref[...], 0.0)


def _tc_mid(psum, pcnt, x, w_lin, b_lin, w_agg, b_agg):
    return pl.pallas_call(
        _tc_mid_body,
        out_shape=(
            jax.ShapeDtypeStruct((N, D), _f32),
            jax.ShapeDtypeStruct((N, D), _f32),
        ),
    )(psum, pcnt, x, w_lin, b_lin, w_agg, b_agg)


def _tc_final_body(ps_ref, pc_ref, hr_ref, wl_ref, bl_ref, w1_ref, b1_ref,
                   w2_ref, b2_ref, o_ref):
    h = _combine_update(ps_ref, pc_ref, _dot_t(hr_ref[...], wl_ref[...]),
                        bl_ref[...])
    t = _dot_t(h, w1_ref[...]) + b1_ref[...]
    y = _dot_t(t, w2_ref[...]) + b2_ref[...]
    m = jnp.max(y, axis=1, keepdims=True)
    z = y - m
    o_ref[...] = z - jnp.log(jnp.sum(jnp.exp(z), axis=1, keepdims=True))


def _tc_final(psum, pcnt, hr, w_lin, b_lin, w1, b1, w2, b2):
    return pl.pallas_call(
        _tc_final_body, out_shape=jax.ShapeDtypeStruct((N, D), _f32)
    )(psum, pcnt, hr, w_lin, b_lin, w1, b1, w2, b2)


def kernel(x, edge_index, batch, W_agg0, b_agg0, W_lin0, b_lin0, W_agg1,
           b_agg1, W_lin1, b_lin1, W_post1, b_post1, W_post2, b_post2):
    src = edge_index[0]
    dst = edge_index[1]
    pad = E_PAD - E
    srcp = jnp.concatenate([src, jnp.zeros((pad,), jnp.int32)])
    dstp = jnp.concatenate([dst, jnp.full((pad,), N, jnp.int32)])
    b_agg0 = b_agg0.reshape(1, D)
    b_lin0 = b_lin0.reshape(1, D)
    b_agg1 = b_agg1.reshape(1, D)
    b_lin1 = b_lin1.reshape(1, D)
    b_post1 = b_post1.reshape(1, D)
    b_post2 = b_post2.reshape(1, D)

    out0 = _tc_pre(x, W_agg0, b_agg0)
    psum0, pcnt = _sc_seg_cnt(out0, srcp, dstp)
    hr, out1 = _tc_mid(psum0, pcnt, x, W_lin0, b_lin0, W_agg1, b_agg1)
    psum1 = _sc_seg(out1, srcp, dstp)
    return _tc_final(psum1, pcnt, hr, W_lin1, b_lin1, W_post1, b_post1,
                     W_post2, b_post2)


# speedup vs baseline: 2.7587x; 1.0564x over previous
import functools

import jax
import jax.numpy as jnp
from jax import lax
from jax.experimental import pallas as pl
from jax.experimental.pallas import tpu as pltpu
from jax.experimental.pallas import tpu_sc as plsc

N = 10000
E = 320000
D = 128
NC = 2
NS = 16
NW = NC * NS
CHUNK = 128
EPW = 10240
E_PAD = NW * EPW
NP8 = N + 8
CW = 16

_f32 = jnp.float32
_MESH = plsc.VectorSubcoreMesh(
    core_axis_name="c", subcore_axis_name="s", num_cores=NC, num_subcores=NS
)


def _make_sc_segment_sum(with_counts, ec):
    cpw = EPW // ec
    assert cpw % 2 == 0
    out_types = [jax.ShapeDtypeStruct((NC, NP8, D), _f32)]
    scratch = [
        pltpu.VMEM((ec,), jnp.int32),
        pltpu.VMEM((ec,), jnp.int32),
        pltpu.VMEM((ec,), jnp.int32),
        pltpu.VMEM((ec,), jnp.int32),
        pltpu.VMEM((ec, D), _f32),
        pltpu.VMEM((ec, D), _f32),
        pltpu.VMEM_SHARED((NP8, D), _f32),
        pltpu.SemaphoreType.DMA,
        pltpu.SemaphoreType.DMA,
    ]
    if with_counts:
        out_types.append(jax.ShapeDtypeStruct((NC, NP8, D), _f32))

    def body(*refs):
        if with_counts:
            (table, srcp, dstp, psum, pcnt, sidx0, didx0, sidx1, didx1,
             rows0, rows1, acc, sem0, sem1) = refs
        else:
            (table, srcp, dstp, psum, sidx0, didx0, sidx1, didx1,
             rows0, rows1, acc, sem0, sem1) = refs
        c = lax.axis_index("c")
        s = lax.axis_index("s")
        w = c * NS + s
        zero16 = jnp.zeros((16,), _f32)
        one16 = jnp.ones((16,), _f32)

        @pl.loop(0, ec)
        def _(i):
            @pl.loop(0, D // 16)
            def _(j):
                rows0[i, pl.ds(j * 16, 16)] = zero16

        def inner():
            zstart = s * 624
            nrows_lo, nrows_hi = 624, 648
            def zero_stripe(zsrc, nrows):
                done = 0
                while done < nrows:
                    step = min(ec, nrows - done)
                    pltpu.sync_copy(zsrc.at[pl.ds(0, step)],
                                    acc.at[pl.ds(zstart + done, step)])
                    done += step
            def zero_mine(zsrc):
                @pl.when(s < NS - 1)
                def _():
                    zero_stripe(zsrc, nrows_lo)
                @pl.when(s == NS - 1)
                def _():
                    zero_stripe(zsrc, nrows_hi)

            def drain_stripe(out_hbm, nrows):
                done = 0
                while done < nrows:
                    step = min(ec, nrows - done)
                    pltpu.sync_copy(acc.at[pl.ds(zstart + done, step)],
                                    rows0.at[pl.ds(0, step)])
                    pltpu.sync_copy(rows0.at[pl.ds(0, step)],
                                    out_hbm.at[c, pl.ds(zstart + done, step)])
                    done += step

            def drain_mine(out_hbm):
                @pl.when(s < NS - 1)
                def _():
                    drain_stripe(out_hbm, nrows_lo)
                @pl.when(s == NS - 1)
                def _():
                    drain_stripe(out_hbm, nrows_hi)

            zero_mine(rows0)
            plsc.subcore_barrier()

            base = w * EPW

            @pl.loop(0, cpw // 2)
            def _(t):
                off0 = base + (2 * t) * ec
                off1 = off0 + ec
                pltpu.sync_copy(srcp.at[pl.ds(off0, ec)], sidx0)
                pltpu.sync_copy(dstp.at[pl.ds(off0, ec)], didx0)
                g0 = pltpu.async_copy(table.at[sidx0], rows0, sem0)
                pltpu.sync_copy(srcp.at[pl.ds(off1, ec)], sidx1)
                pltpu.sync_copy(dstp.at[pl.ds(off1, ec)], didx1)
                g1 = pltpu.async_copy(table.at[sidx1], rows1, sem1)
                g0.wait()
                pltpu.sync_copy(rows0, acc.at[didx0], add=True)
                g1.wait()
                pltpu.sync_copy(rows1, acc.at[didx1], add=True)

            plsc.subcore_barrier()
            drain_mine(psum)

            if with_counts:
                @pl.loop(0, ec)
                def _(i):
                    @pl.loop(0, D // 16)
                    def _(j):
                        rows0[i, pl.ds(j * 16, 16)] = zero16
                        rows1[i, pl.ds(j * 16, 16)] = one16

                zero_mine(rows0)
                plsc.subcore_barrier()

                @pl.loop(0, cpw // 2)
                def _(t):
                    off0 = base + (2 * t) * ec
                    pltpu.sync_copy(dstp.at[pl.ds(off0, ec)], didx0)
                    pltpu.sync_copy(rows1, acc.at[didx0], add=True)
                    pltpu.sync_copy(dstp.at[pl.ds(off0 + ec, ec)], didx1)
                    pltpu.sync_copy(rows1, acc.at[didx1], add=True)

                plsc.subcore_barrier()
                drain_mine(pcnt)

        inner()

    return pl.kernel(
        body,
        out_type=tuple(out_types) if with_counts else out_types[0],
        mesh=_MESH,
        scratch_types=scratch,
    )


_sc_seg_cnt = _make_sc_segment_sum(with_counts=True, ec=64)
_sc_seg = _make_sc_segment_sum(with_counts=False, ec=64)

_DOT = dict(precision=lax.Precision.HIGHEST, preferred_element_type=_f32)


def _dot_t(a, w):
    return lax.dot_general(a, w, (((1,), (1,)), ((), ())), **_DOT)


def _tc_pre_body(x_ref, w_ref, b_ref, o_ref):
    o_ref[...] = jnp.maximum(_dot_t(x_ref[...], w_ref[...]) + b_ref[...], 0.0)


def _tc_pre(x, w, b):
    return pl.pallas_call(
        _tc_pre_body, out_shape=jax.ShapeDtypeStruct((N, D), _f32)
    )(x, w, b)


def _combine_update(ps_ref, pc_ref, x_lin, b):
    cnt = (pc_ref[0, pl.ds(0, N), 0:1] + pc_ref[1, pl.ds(0, N), 0:1])
    agg = ((ps_ref[0, pl.ds(0, N)] + ps_ref[1, pl.ds(0, N)])
           / jnp.maximum(cnt, 1.0))
    h = jnp.maximum(agg + x_lin + b, 0.0)
    nrm = jnp.sqrt(jnp.sum(h * h, axis=1, keepdims=True))
    h = h / jnp.maximum(nrm, 1e-12)
    return jnp.maximum(h, 0.0)


def _tc_mid_body(ps_ref, pc_ref, x_ref, wl_ref, bl_ref, wa_ref, ba_ref,
                 hr_ref, o1_ref):
    h = _combine_update(ps_ref, pc_ref, _dot_t(x_ref[...], wl_ref[...]),
                        bl_ref[...])
    hr_ref[...] = h
    o1_ref[...] = jnp.maximum(_dot_t(h, wa_ref[...]) + ba_ref[...], 0.0)


def _tc_mid(psum, pcnt, x, w_lin, b_lin, w_agg, b_agg):
    return pl.pallas_call(
        _tc_mid_body,
        out_shape=(
            jax.ShapeDtypeStruct((N, D), _f32),
            jax.ShapeDtypeStruct((N, D), _f32),
        ),
    )(psum, pcnt, x, w_lin, b_lin, w_agg, b_agg)


def _tc_final_body(ps_ref, pc_ref, hr_ref, wl_ref, bl_ref, w1_ref, b1_ref,
                   w2_ref, b2_ref, o_ref):
    h = _combine_update(ps_ref, pc_ref, _dot_t(hr_ref[...], wl_ref[...]),
                        bl_ref[...])
    t = _dot_t(h, w1_ref[...]) + b1_ref[...]
    y = _dot_t(t, w2_ref[...]) + b2_ref[...]
    m = jnp.max(y, axis=1, keepdims=True)
    z = y - m
    o_ref[...] = z - jnp.log(jnp.sum(jnp.exp(z), axis=1, keepdims=True))


def _tc_final(psum, pcnt, hr, w_lin, b_lin, w1, b1, w2, b2):
    return pl.pallas_call(
        _tc_final_body, out_shape=jax.ShapeDtypeStruct((N, D), _f32)
    )(psum, pcnt, hr, w_lin, b_lin, w1, b1, w2, b2)


def kernel(x, edge_index, batch, W_agg0, b_agg0, W_lin0, b_lin0, W_agg1,
           b_agg1, W_lin1, b_lin1, W_post1, b_post1, W_post2, b_post2):
    src = edge_index[0]
    dst = edge_index[1]
    pad = E_PAD - E
    srcp = jnp.concatenate([src, jnp.zeros((pad,), jnp.int32)])
    dstp = jnp.concatenate([dst, jnp.full((pad,), N, jnp.int32)])
    b_agg0 = b_agg0.reshape(1, D)
    b_lin0 = b_lin0.reshape(1, D)
    b_agg1 = b_agg1.reshape(1, D)
    b_lin1 = b_lin1.reshape(1, D)
    b_post1 = b_post1.reshape(1, D)
    b_post2 = b_post2.reshape(1, D)

    out0 = _tc_pre(x, W_agg0, b_agg0)
    psum0, pcnt = _sc_seg_cnt(out0, srcp, dstp)
    hr, out1 = _tc_mid(psum0, pcnt, x, W_lin0, b_lin0, W_agg1, b_agg1)
    psum1 = _sc_seg(out1, srcp, dstp)
    return _tc_final(psum1, pcnt, hr, W_lin1, b_lin1, W_post1, b_post1,
                     W_post2, b_post2)
